# Initial kernel scaffold; baseline (speedup 1.0000x reference)
import jax
import jax.numpy as jnp
from jax.experimental import pallas as pl


def kernel(z, edge_index, W1_w, W1_b, W2_w, W2_b):
    raise NotImplementedError("write your pallas kernel here")


# speedup vs baseline: 1.2723x; 1.2723x over previous
import functools

import jax
import jax.numpy as jnp
from jax import lax
from jax.experimental import pallas as pl
from jax.experimental.pallas import tpu as pltpu
from jax.experimental.pallas import tpu_sc as plsc

N_NODES = 10000
N_EDGES = 320000
H = 128

NC = 2
NS = 16
NW = NC * NS

TOTAL_ROWS = 2 * N_EDGES
CHUNK = 128
ROWS_PER_W = 20480
PAD_ROWS = ROWS_PER_W * NW
N_CHUNKS = ROWS_PER_W // CHUNK


def _sc_gather_body(z_hbm, idx_hbm, out_hbm, idx_v, rows_v, sem):
    wid = lax.axis_index("s") * NC + lax.axis_index("c")
    base0 = wid * ROWS_PER_W

    @pl.loop(0, N_CHUNKS)
    def _(i):
        base = base0 + i * CHUNK
        pltpu.sync_copy(idx_hbm.at[pl.ds(base, CHUNK)], idx_v)
        pltpu.async_copy(z_hbm.at[idx_v], rows_v, sem).wait()
        pltpu.sync_copy(rows_v, out_hbm.at[pl.ds(base, CHUNK)])


_sc_gather = pl.kernel(
    _sc_gather_body,
    out_type=jax.ShapeDtypeStruct((PAD_ROWS, H), jnp.float32),
    mesh=plsc.VectorSubcoreMesh(core_axis_name="c", subcore_axis_name="s",
                                num_cores=NC, num_subcores=NS),
    scratch_types=[
        pltpu.VMEM((CHUNK,), jnp.int32),
        pltpu.VMEM((CHUNK, H), jnp.float32),
        pltpu.SemaphoreType.DMA,
    ],
)

BE = 2560
NB = N_EDGES // BE
OUT_ROWS = BE // H


def _mlp_body(zs_ref, zd_ref, a_ref, b_ref, b1_ref, w2_ref, b2_ref, o_ref):
    zs = zs_ref[...]
    zd = zd_ref[...]
    avg = (zs + zd) * 0.5
    dif = zs - zd
    var = dif * dif
    dn = (((1,), (1,)), ((), ()))
    h1 = lax.dot_general(avg, a_ref[...], dn, preferred_element_type=jnp.float32)
    h1 = h1 + lax.dot_general(var, b_ref[...], dn, preferred_element_type=jnp.float32)
    h1 = jnp.maximum(h1 + b1_ref[...], 0.0)
    logit = jnp.sum(h1 * w2_ref[...], axis=1) + b2_ref[0, 0]
    o_ref[...] = jax.nn.sigmoid(logit).reshape(1, OUT_ROWS, H)


def _tc_mlp(gathered, a, b, b1, w2, b2):
    return pl.pallas_call(
        _mlp_body,
        grid=(NB,),
        in_specs=[
            pl.BlockSpec((BE, H), lambda i: (i, 0)),
            pl.BlockSpec((BE, H), lambda i: (i + NB, 0)),
            pl.BlockSpec((H, H), lambda i: (0, 0)),
            pl.BlockSpec((H, H), lambda i: (0, 0)),
            pl.BlockSpec((1, H), lambda i: (0, 0)),
            pl.BlockSpec((1, H), lambda i: (0, 0)),
            pl.BlockSpec((1, 1), lambda i: (0, 0), memory_space=pltpu.SMEM),
        ],
        out_specs=pl.BlockSpec((1, OUT_ROWS, H), lambda i: (i, 0, 0)),
        out_shape=jax.ShapeDtypeStruct((NB, OUT_ROWS, H), jnp.float32),
    )(gathered, gathered, a, b, b1, w2, b2)


def kernel(z, edge_index, W1_w, W1_b, W2_w, W2_b):
    ei = edge_index.astype(jnp.int32)
    pad = jnp.zeros((PAD_ROWS - TOTAL_ROWS,), jnp.int32)
    idx_all = jnp.concatenate([ei[0], ei[1], pad], axis=0)
    gathered = _sc_gather(z, idx_all)
    a = W1_w[:, :H]
    b = W1_w[:, H:]
    out2d = _tc_mlp(gathered, a, b, W1_b.reshape(1, H),
                    W2_w.reshape(1, H), W2_b.reshape(1, 1))
    return out2d.reshape(N_EDGES)


# speedup vs baseline: 1.3338x; 1.0484x over previous
import functools

import jax
import jax.numpy as jnp
from jax import lax
from jax.experimental import pallas as pl
from jax.experimental.pallas import tpu as pltpu
from jax.experimental.pallas import tpu_sc as plsc

N_NODES = 10000
N_EDGES = 320000
H = 128

NC = 2
NS = 16
NW = NC * NS

TOTAL_ROWS = 2 * N_EDGES
CHUNK = 128
ROWS_PER_W = 20480
PAD_ROWS = ROWS_PER_W * NW
N_CHUNKS = ROWS_PER_W // CHUNK
K = 2
GROUP = K * CHUNK
NGROUP = ROWS_PER_W // GROUP
NBUF = 2


def _sc_gather_body(z_hbm, idx_hbm, out_hbm, idx_v, rows_v,
                    sem_g0, sem_g1, sem_w0, sem_w1):
    sem_g = (sem_g0, sem_g1)
    sem_w = (sem_w0, sem_w1)
    wid = lax.axis_index("s") * NC + lax.axis_index("c")
    row0 = wid * ROWS_PER_W
    pltpu.sync_copy(idx_hbm.at[pl.ds(wid * N_CHUNKS, N_CHUNKS)], idx_v)

    @pl.loop(0, NGROUP, step=NBUF)
    def _(g0):
        for b in range(NBUF):
            g = g0 + b

            @pl.when(g >= NBUF)
            def _():
                prev = g - NBUF
                pltpu.make_async_copy(
                    rows_v.at[b],
                    out_hbm.at[pl.ds(row0 + prev * GROUP, GROUP)],
                    sem_w[b]).wait()

            for j in range(K):
                pltpu.async_copy(
                    z_hbm.at[idx_v.at[g * K + j]],
                    rows_v.at[b, pl.ds(j * CHUNK, CHUNK)],
                    sem_g[b])
            pltpu.make_async_copy(z_hbm.at[pl.ds(0, GROUP)], rows_v.at[b],
                                  sem_g[b]).wait()
            pltpu.async_copy(rows_v.at[b],
                             out_hbm.at[pl.ds(row0 + g * GROUP, GROUP)],
                             sem_w[b])

    for b in range(NBUF):
        last = NGROUP - NBUF + b
        pltpu.make_async_copy(
            rows_v.at[b],
            out_hbm.at[pl.ds(row0 + last * GROUP, GROUP)],
            sem_w[b]).wait()


_sc_gather = pl.kernel(
    _sc_gather_body,
    out_type=jax.ShapeDtypeStruct((PAD_ROWS, H), jnp.float32),
    mesh=plsc.VectorSubcoreMesh(core_axis_name="c", subcore_axis_name="s",
                                num_cores=NC, num_subcores=NS),
    scratch_types=[
        pltpu.VMEM((N_CHUNKS, CHUNK), jnp.int32),
        pltpu.VMEM((NBUF, GROUP, H), jnp.float32),
        pltpu.SemaphoreType.DMA,
        pltpu.SemaphoreType.DMA,
        pltpu.SemaphoreType.DMA,
        pltpu.SemaphoreType.DMA,
    ],
)

BE = 2560
NB = N_EDGES // BE
OUT_ROWS = BE // H


def _mlp_body(zs_ref, zd_ref, a_ref, b_ref, b1_ref, w2_ref, b2_ref, o_ref):
    zs = zs_ref[...]
    zd = zd_ref[...]
    avg = (zs + zd) * 0.5
    dif = zs - zd
    var = dif * dif
    dn = (((1,), (1,)), ((), ()))
    h1 = lax.dot_general(avg, a_ref[...], dn, preferred_element_type=jnp.float32)
    h1 = h1 + lax.dot_general(var, b_ref[...], dn, preferred_element_type=jnp.float32)
    h1 = jnp.maximum(h1 + b1_ref[...], 0.0)
    logit = jnp.sum(h1 * w2_ref[...], axis=1) + b2_ref[0, 0]
    o_ref[...] = jax.nn.sigmoid(logit).reshape(1, OUT_ROWS, H)


def _tc_mlp(gathered, a, b, b1, w2, b2):
    return pl.pallas_call(
        _mlp_body,
        grid=(NB,),
        in_specs=[
            pl.BlockSpec((BE, H), lambda i: (i, 0)),
            pl.BlockSpec((BE, H), lambda i: (i + NB, 0)),
            pl.BlockSpec((H, H), lambda i: (0, 0)),
            pl.BlockSpec((H, H), lambda i: (0, 0)),
            pl.BlockSpec((1, H), lambda i: (0, 0)),
            pl.BlockSpec((1, H), lambda i: (0, 0)),
            pl.BlockSpec((1, 1), lambda i: (0, 0), memory_space=pltpu.SMEM),
        ],
        out_specs=pl.BlockSpec((1, OUT_ROWS, H), lambda i: (i, 0, 0)),
        out_shape=jax.ShapeDtypeStruct((NB, OUT_ROWS, H), jnp.float32),
    )(gathered, gathered, a, b, b1, w2, b2)


def kernel(z, edge_index, W1_w, W1_b, W2_w, W2_b):
    ei = edge_index.astype(jnp.int32)
    pad = jnp.zeros((PAD_ROWS - TOTAL_ROWS,), jnp.int32)
    idx_all = jnp.concatenate([ei[0], ei[1], pad], axis=0)
    idx_2d = idx_all.reshape(PAD_ROWS // CHUNK, CHUNK)
    gathered = _sc_gather(z, idx_2d)
    a = W1_w[:, :H]
    b = W1_w[:, H:]
    out2d = _tc_mlp(gathered, a, b, W1_b.reshape(1, H),
                    W2_w.reshape(1, H), W2_b.reshape(1, 1))
    return out2d.reshape(N_EDGES)


# speedup vs baseline: 4.7064x; 3.5284x over previous
import jax
import jax.numpy as jnp
from jax import lax
from jax.experimental import pallas as pl
from jax.experimental.pallas import tpu as pltpu
from jax.experimental.pallas import tpu_sc as plsc

N_NODES = 10000
N_EDGES = 320000
H = 128

NC = 2
NS = 16
NW = NC * NS

TOTAL_ROWS = 2 * N_EDGES
CHUNK = 128
ROWS_PER_W = 20480
PAD_ROWS = ROWS_PER_W * NW
N_CHUNKS = ROWS_PER_W // CHUNK
NPHASE = 4
IDXBUF = N_CHUNKS // NPHASE
NBUF = 2


def _sc_gather_body(z_hbm, idx_hbm, out_hbm, z_sh, idx_v, rows_v,
                    sem_g0, sem_g1, sem_w0, sem_w1):
    sem_g = (sem_g0, sem_g1)
    sem_w = (sem_w0, sem_w1)
    sid = lax.axis_index("s")
    wid = sid * NC + lax.axis_index("c")
    row0 = wid * ROWS_PER_W

    @pl.when(sid == 0)
    def _():
        pltpu.sync_copy(z_hbm, z_sh)

    plsc.subcore_barrier()

    for p in range(NPHASE):
        pltpu.sync_copy(
            idx_hbm.at[pl.ds(wid * N_CHUNKS + p * IDXBUF, IDXBUF)], idx_v)

        @pl.loop(0, IDXBUF, step=NBUF)
        def _(g0):
            for b in range(NBUF):
                g = g0 + b
                gg = p * IDXBUF + g

                @pl.when(gg >= NBUF)
                def _():
                    prev = gg - NBUF
                    pltpu.make_async_copy(
                        rows_v.at[b],
                        out_hbm.at[pl.ds(row0 + prev * CHUNK, CHUNK)],
                        sem_w[b]).wait()

                pltpu.async_copy(z_sh.at[idx_v.at[g]], rows_v.at[b],
                                 sem_g[b])
                pltpu.make_async_copy(z_hbm.at[pl.ds(0, CHUNK)],
                                      rows_v.at[b], sem_g[b]).wait()
                pltpu.async_copy(rows_v.at[b],
                                 out_hbm.at[pl.ds(row0 + gg * CHUNK, CHUNK)],
                                 sem_w[b])

    for b in range(NBUF):
        last = N_CHUNKS - NBUF + b
        pltpu.make_async_copy(
            rows_v.at[b],
            out_hbm.at[pl.ds(row0 + last * CHUNK, CHUNK)],
            sem_w[b]).wait()


_sc_gather = pl.kernel(
    _sc_gather_body,
    out_type=jax.ShapeDtypeStruct((PAD_ROWS, H), jnp.float32),
    mesh=plsc.VectorSubcoreMesh(core_axis_name="c", subcore_axis_name="s",
                                num_cores=NC, num_subcores=NS),
    scratch_types=[
        pltpu.VMEM_SHARED((N_NODES, H), jnp.float32),
        pltpu.VMEM((IDXBUF, CHUNK), jnp.int32),
        pltpu.VMEM((NBUF, CHUNK, H), jnp.float32),
        pltpu.SemaphoreType.DMA,
        pltpu.SemaphoreType.DMA,
        pltpu.SemaphoreType.DMA,
        pltpu.SemaphoreType.DMA,
    ],
)

BE = 2560
NB = N_EDGES // BE
OUT_ROWS = BE // H


def _mlp_body(zs_ref, zd_ref, a_ref, b_ref, b1_ref, w2_ref, b2_ref, o_ref):
    zs = zs_ref[...]
    zd = zd_ref[...]
    avg = (zs + zd) * 0.5
    dif = zs - zd
    var = dif * dif
    dn = (((1,), (1,)), ((), ()))
    h1 = lax.dot_general(avg, a_ref[...], dn, preferred_element_type=jnp.float32)
    h1 = h1 + lax.dot_general(var, b_ref[...], dn, preferred_element_type=jnp.float32)
    h1 = jnp.maximum(h1 + b1_ref[...], 0.0)
    logit = jnp.sum(h1 * w2_ref[...], axis=1) + b2_ref[0, 0]
    o_ref[...] = jax.nn.sigmoid(logit).reshape(1, OUT_ROWS, H)


def _tc_mlp(gathered, a, b, b1, w2, b2):
    return pl.pallas_call(
        _mlp_body,
        grid=(NB,),
        in_specs=[
            pl.BlockSpec((BE, H), lambda i: (i, 0)),
            pl.BlockSpec((BE, H), lambda i: (i + NB, 0)),
            pl.BlockSpec((H, H), lambda i: (0, 0)),
            pl.BlockSpec((H, H), lambda i: (0, 0)),
            pl.BlockSpec((1, H), lambda i: (0, 0)),
            pl.BlockSpec((1, H), lambda i: (0, 0)),
            pl.BlockSpec((1, 1), lambda i: (0, 0), memory_space=pltpu.SMEM),
        ],
        out_specs=pl.BlockSpec((1, OUT_ROWS, H), lambda i: (i, 0, 0)),
        out_shape=jax.ShapeDtypeStruct((NB, OUT_ROWS, H), jnp.float32),
    )(gathered, gathered, a, b, b1, w2, b2)


def kernel(z, edge_index, W1_w, W1_b, W2_w, W2_b):
    ei = edge_index.astype(jnp.int32)
    pad = jnp.zeros((PAD_ROWS - TOTAL_ROWS,), jnp.int32)
    idx_all = jnp.concatenate([ei[0], ei[1], pad], axis=0)
    idx_2d = idx_all.reshape(PAD_ROWS // CHUNK, CHUNK)
    gathered = _sc_gather(z, idx_2d)
    a = W1_w[:, :H]
    b = W1_w[:, H:]
    out2d = _tc_mlp(gathered, a, b, W1_b.reshape(1, H),
                    W2_w.reshape(1, H), W2_b.reshape(1, 1))
    return out2d.reshape(N_EDGES)


# speedup vs baseline: 5.5591x; 1.1812x over previous
import jax
import jax.numpy as jnp
from jax import lax
from jax.experimental import pallas as pl
from jax.experimental.pallas import tpu as pltpu
from jax.experimental.pallas import tpu_sc as plsc

N_NODES = 10000
N_EDGES = 320000
H = 128

NC = 2
NS = 16
NW = NC * NS

NHALF = 2
EH = N_EDGES // NHALF
TOTAL_ROWS = 2 * EH
CHUNK = 128
ROWS_PER_W = 10240
PAD_ROWS = ROWS_PER_W * NW
N_CHUNKS = ROWS_PER_W // CHUNK
NBUF = 2


def _sc_gather_body(z_hbm, idx_hbm, out_hbm, z_sh, idx_v, rows_v,
                    sem_g0, sem_g1, sem_w0, sem_w1):
    sem_g = (sem_g0, sem_g1)
    sem_w = (sem_w0, sem_w1)
    sid = lax.axis_index("s")
    wid = sid * NC + lax.axis_index("c")
    row0 = wid * ROWS_PER_W

    @pl.when(sid == 0)
    def _():
        pltpu.sync_copy(z_hbm, z_sh)

    pltpu.sync_copy(idx_hbm.at[pl.ds(wid * N_CHUNKS, N_CHUNKS)], idx_v)
    plsc.subcore_barrier()

    @pl.loop(0, N_CHUNKS, step=NBUF)
    def _(g0):
        for b in range(NBUF):
            g = g0 + b

            @pl.when(g >= NBUF)
            def _():
                prev = g - NBUF
                pltpu.make_async_copy(
                    rows_v.at[b],
                    out_hbm.at[pl.ds(row0 + prev * CHUNK, CHUNK)],
                    sem_w[b]).wait()

            pltpu.async_copy(z_sh.at[idx_v.at[g]], rows_v.at[b], sem_g[b])
            pltpu.make_async_copy(z_hbm.at[pl.ds(0, CHUNK)],
                                  rows_v.at[b], sem_g[b]).wait()
            pltpu.async_copy(rows_v.at[b],
                             out_hbm.at[pl.ds(row0 + g * CHUNK, CHUNK)],
                             sem_w[b])

    for b in range(NBUF):
        last = N_CHUNKS - NBUF + b
        pltpu.make_async_copy(
            rows_v.at[b],
            out_hbm.at[pl.ds(row0 + last * CHUNK, CHUNK)],
            sem_w[b]).wait()


_sc_gather = pl.kernel(
    _sc_gather_body,
    out_type=jax.ShapeDtypeStruct((PAD_ROWS, H), jnp.float32),
    mesh=plsc.VectorSubcoreMesh(core_axis_name="c", subcore_axis_name="s",
                                num_cores=NC, num_subcores=NS),
    scratch_types=[
        pltpu.VMEM_SHARED((N_NODES, H), jnp.float32),
        pltpu.VMEM((N_CHUNKS, CHUNK), jnp.int32),
        pltpu.VMEM((NBUF, CHUNK, H), jnp.float32),
        pltpu.SemaphoreType.DMA,
        pltpu.SemaphoreType.DMA,
        pltpu.SemaphoreType.DMA,
        pltpu.SemaphoreType.DMA,
    ],
)

BE = 6400
NB = EH // BE
OUT_ROWS = BE // H


def _mlp_body(zs_ref, zd_ref, a_ref, b_ref, b1_ref, w2_ref, b2_ref, o_ref):
    zs = zs_ref[...]
    zd = zd_ref[...]
    avg = (zs + zd) * 0.5
    dif = zs - zd
    var = dif * dif
    dn = (((1,), (1,)), ((), ()))
    h1 = lax.dot_general(avg, a_ref[...], dn, preferred_element_type=jnp.float32)
    h1 = h1 + lax.dot_general(var, b_ref[...], dn, preferred_element_type=jnp.float32)
    h1 = jnp.maximum(h1 + b1_ref[...], 0.0)
    logit = jnp.sum(h1 * w2_ref[...], axis=1) + b2_ref[0, 0]
    o_ref[...] = jax.nn.sigmoid(logit).reshape(1, OUT_ROWS, H)


def _tc_mlp(gathered, a, b, b1, w2, b2):
    return pl.pallas_call(
        _mlp_body,
        grid=(NB,),
        in_specs=[
            pl.BlockSpec((BE, H), lambda i: (i, 0)),
            pl.BlockSpec((BE, H), lambda i: (i + NB, 0)),
            pl.BlockSpec((H, H), lambda i: (0, 0)),
            pl.BlockSpec((H, H), lambda i: (0, 0)),
            pl.BlockSpec((1, H), lambda i: (0, 0)),
            pl.BlockSpec((1, H), lambda i: (0, 0)),
            pl.BlockSpec((1, 1), lambda i: (0, 0), memory_space=pltpu.SMEM),
        ],
        out_specs=pl.BlockSpec((1, OUT_ROWS, H), lambda i: (i, 0, 0)),
        out_shape=jax.ShapeDtypeStruct((NB, OUT_ROWS, H), jnp.float32),
    )(gathered, gathered, a, b, b1, w2, b2)


def kernel(z, edge_index, W1_w, W1_b, W2_w, W2_b):
    ei = edge_index.astype(jnp.int32)
    pad = jnp.zeros((PAD_ROWS - TOTAL_ROWS,), jnp.int32)
    a = W1_w[:, :H]
    b = W1_w[:, H:]
    b1 = W1_b.reshape(1, H)
    w2 = W2_w.reshape(1, H)
    b2 = W2_b.reshape(1, 1)
    outs = []
    for h in range(NHALF):
        src = lax.slice_in_dim(ei[0], h * EH, (h + 1) * EH)
        dst = lax.slice_in_dim(ei[1], h * EH, (h + 1) * EH)
        idx_all = jnp.concatenate([src, dst, pad], axis=0)
        idx_2d = idx_all.reshape(PAD_ROWS // CHUNK, CHUNK)
        gathered = _sc_gather(z, idx_2d)
        outs.append(_tc_mlp(gathered, a, b, b1, w2, b2).reshape(EH))
    return jnp.concatenate(outs, axis=0)
